# Initial kernel scaffold; baseline (speedup 1.0000x reference)
import functools

import jax
import jax.numpy as jnp
from jax import lax
from jax.experimental import pallas as pl
from jax.experimental.pallas import tpu as pltpu
from jax.experimental.pallas import tpu_sc as plsc

NUM_RELATIONS = 1000
EMBED_DIM = 64
HIDDEN = 128
B, S = 1024, 20
N = B * S
NC, NS = 2, 16
NW = NC * NS
ROWS_PER_W = N // NW
R_CHUNK = 4
IDX_PER_CHUNK = R_CHUNK * S
N_CHUNKS = ROWS_PER_W // R_CHUNK


def _sc_gather_mean(idx_flat, table):
    mesh = plsc.VectorSubcoreMesh(core_axis_name="c", subcore_axis_name="s")

    @functools.partial(
        pl.kernel,
        out_type=jax.ShapeDtypeStruct((N, EMBED_DIM), jnp.float32),
        mesh=mesh,
        scratch_types=[
            pltpu.VMEM((IDX_PER_CHUNK,), jnp.int32),
            pltpu.VMEM((IDX_PER_CHUNK, EMBED_DIM), jnp.float32),
            pltpu.VMEM((R_CHUNK, EMBED_DIM), jnp.float32),
            pltpu.SemaphoreType.DMA,
        ],
    )
    def k(idx_hbm, table_hbm, agg_hbm, idx_v, rows_v, out_v, sem):
        wid = lax.axis_index("s") * NC + lax.axis_index("c")
        row0 = wid * ROWS_PER_W

        def body(g, carry):
            rbase = row0 + g * R_CHUNK
            pltpu.sync_copy(idx_hbm.at[pl.ds(rbase * S, IDX_PER_CHUNK)], idx_v)
            pltpu.async_copy(table_hbm.at[idx_v], rows_v, sem).wait()
            for rr in range(R_CHUNK):
                for c in range(EMBED_DIM // 16):
                    acc = rows_v[rr * S, pl.ds(c * 16, 16)]
                    for j in range(1, S):
                        acc = acc + rows_v[rr * S + j, pl.ds(c * 16, 16)]
                    out_v[rr, pl.ds(c * 16, 16)] = acc * (1.0 / S)
            pltpu.sync_copy(out_v, agg_hbm.at[pl.ds(rbase, R_CHUNK)])
            return carry

        lax.fori_loop(0, N_CHUNKS, body, None)

    return k(idx_flat, table)


def _mlp(agg, W1, b1, W2, b2):
    ROWS_BLK = 2048

    def body(a_ref, w1_ref, b1_ref, w2_ref, b2_ref, o_ref):
        h = jnp.dot(a_ref[...], w1_ref[...], preferred_element_type=jnp.float32)
        h = jnp.maximum(h + b1_ref[...], 0.0)
        o_ref[...] = (
            jnp.dot(h, w2_ref[...], preferred_element_type=jnp.float32)
            + b2_ref[...]
        )

    return pl.pallas_call(
        body,
        grid=(N // ROWS_BLK,),
        in_specs=[
            pl.BlockSpec((ROWS_BLK, EMBED_DIM), lambda i: (i, 0)),
            pl.BlockSpec((EMBED_DIM, HIDDEN), lambda i: (0, 0)),
            pl.BlockSpec((1, HIDDEN), lambda i: (0, 0)),
            pl.BlockSpec((HIDDEN, HIDDEN), lambda i: (0, 0)),
            pl.BlockSpec((1, HIDDEN), lambda i: (0, 0)),
        ],
        out_specs=pl.BlockSpec((ROWS_BLK, HIDDEN), lambda i: (i, 0)),
        out_shape=jax.ShapeDtypeStruct((N, HIDDEN), jnp.float32),
    )(agg, W1, b1, W2, b2)


def kernel(kg_spatial_matrix, rel_table, W1, b1, W2, b2):
    table = rel_table.at[0].set(0.0)
    idx_flat = kg_spatial_matrix.reshape(-1)
    agg = _sc_gather_mean(idx_flat, table)
    out = _mlp(agg, W1, b1.reshape(1, HIDDEN), W2, b2.reshape(1, HIDDEN))
    return out.reshape(B, S, HIDDEN)


# speedup vs baseline: 5.0153x; 5.0153x over previous
import functools

import jax
import jax.numpy as jnp
from jax import lax
from jax.experimental import pallas as pl
from jax.experimental.pallas import tpu as pltpu
from jax.experimental.pallas import tpu_sc as plsc

NUM_RELATIONS = 1000
EMBED_DIM = 64
HIDDEN = 128
B, S = 1024, 20
N = B * S
NC, NS = 2, 16
NW = NC * NS
ROWS_PER_W = N // NW
R_CHUNK = 4
IDX_PER_CHUNK = R_CHUNK * S
N_CHUNKS = ROWS_PER_W // R_CHUNK


def _sc_gather_mean(idx_flat, table):
    mesh = plsc.VectorSubcoreMesh(core_axis_name="c", subcore_axis_name="s")

    @functools.partial(
        pl.kernel,
        out_type=jax.ShapeDtypeStruct((N, EMBED_DIM), jnp.float32),
        mesh=mesh,
        scratch_types=[
            pltpu.VMEM((IDX_PER_CHUNK,), jnp.int32),
            pltpu.VMEM((IDX_PER_CHUNK, EMBED_DIM), jnp.float32),
            pltpu.VMEM((R_CHUNK, EMBED_DIM), jnp.float32),
            pltpu.SemaphoreType.DMA,
        ],
        compiler_params=pltpu.CompilerParams(use_tc_tiling_on_sc=False),
    )
    def k(idx_hbm, table_hbm, agg_hbm, idx_v, rows_v, out_v, sem):
        wid = lax.axis_index("s") * NC + lax.axis_index("c")
        row0 = wid * ROWS_PER_W

        def body(g, carry):
            rbase = row0 + g * R_CHUNK
            pltpu.sync_copy(idx_hbm.at[pl.ds(rbase * S, IDX_PER_CHUNK)], idx_v)
            pltpu.async_copy(table_hbm.at[idx_v], rows_v, sem).wait()
            for rr in range(R_CHUNK):
                for c in range(EMBED_DIM // 16):
                    acc = rows_v[rr * S, pl.ds(c * 16, 16)]
                    for j in range(1, S):
                        acc = acc + rows_v[rr * S + j, pl.ds(c * 16, 16)]
                    out_v[rr, pl.ds(c * 16, 16)] = acc * (1.0 / S)
            pltpu.sync_copy(out_v, agg_hbm.at[pl.ds(rbase, R_CHUNK)])
            return carry

        lax.fori_loop(0, N_CHUNKS, body, None)

    return k(idx_flat, table)


def _mlp(agg, W1, b1, W2, b2):
    ROWS_BLK = 2048

    def body(a_ref, w1_ref, b1_ref, w2_ref, b2_ref, o_ref):
        h = jnp.dot(a_ref[...], w1_ref[...], preferred_element_type=jnp.float32)
        h = jnp.maximum(h + b1_ref[...], 0.0)
        o_ref[...] = (
            jnp.dot(h, w2_ref[...], preferred_element_type=jnp.float32)
            + b2_ref[...]
        )

    return pl.pallas_call(
        body,
        grid=(N // ROWS_BLK,),
        in_specs=[
            pl.BlockSpec((ROWS_BLK, EMBED_DIM), lambda i: (i, 0)),
            pl.BlockSpec((EMBED_DIM, HIDDEN), lambda i: (0, 0)),
            pl.BlockSpec((1, HIDDEN), lambda i: (0, 0)),
            pl.BlockSpec((HIDDEN, HIDDEN), lambda i: (0, 0)),
            pl.BlockSpec((1, HIDDEN), lambda i: (0, 0)),
        ],
        out_specs=pl.BlockSpec((ROWS_BLK, HIDDEN), lambda i: (i, 0)),
        out_shape=jax.ShapeDtypeStruct((N, HIDDEN), jnp.float32),
    )(agg, W1, b1, W2, b2)


def kernel(kg_spatial_matrix, rel_table, W1, b1, W2, b2):
    table = rel_table.at[0].set(0.0)
    idx_flat = kg_spatial_matrix.reshape(-1)
    agg = _sc_gather_mean(idx_flat, table)
    out = _mlp(agg, W1, b1.reshape(1, HIDDEN), W2, b2.reshape(1, HIDDEN))
    return out.reshape(B, S, HIDDEN)


# speedup vs baseline: 8.6852x; 1.7317x over previous
import functools

import jax
import jax.numpy as jnp
from jax import lax
from jax.experimental import pallas as pl
from jax.experimental.pallas import tpu as pltpu
from jax.experimental.pallas import tpu_sc as plsc

NUM_RELATIONS = 1000
EMBED_DIM = 64
HIDDEN = 128
B, S = 1024, 20
N = B * S
NC, NS = 2, 16
NW = NC * NS
ROWS_PER_W = N // NW
R_CHUNK = 4
IDX_PER_CHUNK = R_CHUNK * S
N_CHUNKS = ROWS_PER_W // R_CHUNK
NBUF = 4


def _sc_gather_mean(idx2d, table):
    mesh = plsc.VectorSubcoreMesh(core_axis_name="c", subcore_axis_name="s")

    @functools.partial(
        pl.kernel,
        out_type=jax.ShapeDtypeStruct((N, EMBED_DIM), jnp.float32),
        mesh=mesh,
        scratch_types=[
            pltpu.VMEM((N_CHUNKS, IDX_PER_CHUNK), jnp.int32),
            [pltpu.VMEM((IDX_PER_CHUNK, EMBED_DIM), jnp.float32)] * NBUF,
            [pltpu.VMEM((R_CHUNK, EMBED_DIM), jnp.float32)] * NBUF,
            [pltpu.SemaphoreType.DMA] * NBUF,
            [pltpu.SemaphoreType.DMA] * NBUF,
        ],
        compiler_params=pltpu.CompilerParams(use_tc_tiling_on_sc=False),
    )
    def k(idx_hbm, table_hbm, agg_hbm, idx_v, rows_v, out_v, gsem, osem):
        wid = lax.axis_index("s") * NC + lax.axis_index("c")
        chunk0 = wid * N_CHUNKS

        pltpu.sync_copy(idx_hbm.at[pl.ds(chunk0 * 1, N_CHUNKS)], idx_v)

        def gather(g, b):
            pltpu.make_async_copy(
                table_hbm.at[idx_v.at[g]], rows_v[b], gsem[b]
            ).start()

        def gather_wait(b):
            pltpu.make_async_copy(
                table_hbm.at[idx_v.at[0]], rows_v[b], gsem[b]
            ).wait()

        def out_start(g, b):
            pltpu.make_async_copy(
                out_v[b], agg_hbm.at[pl.ds((chunk0 + g) * R_CHUNK, R_CHUNK)],
                osem[b],
            ).start()

        def out_wait(b):
            pltpu.make_async_copy(
                out_v[b], agg_hbm.at[pl.ds(0, R_CHUNK)], osem[b]
            ).wait()

        for b in range(NBUF):
            gather(b, b)

        def outer(t, carry):
            for b in range(NBUF):
                g = t * NBUF + b
                gather_wait(b)
                @pl.when(g >= NBUF)
                def _():
                    out_wait(b)

                rv, ov = rows_v[b], out_v[b]
                for rr in range(R_CHUNK):
                    for c in range(EMBED_DIM // 16):
                        acc = rv[rr * S, pl.ds(c * 16, 16)]
                        for j in range(1, S):
                            acc = acc + rv[rr * S + j, pl.ds(c * 16, 16)]
                        ov[rr, pl.ds(c * 16, 16)] = acc * (1.0 / S)
                out_start(g, b)

                @pl.when(g + NBUF < N_CHUNKS)
                def _():
                    gather(g + NBUF, b)

            return carry

        lax.fori_loop(0, N_CHUNKS // NBUF, outer, None)
        for b in range(NBUF):
            out_wait(b)

    return k(idx2d, table)


def _mlp(agg, W1, b1, W2, b2):
    ROWS_BLK = 2048

    def body(a_ref, w1_ref, b1_ref, w2_ref, b2_ref, o_ref):
        h = jnp.dot(a_ref[...], w1_ref[...], preferred_element_type=jnp.float32)
        h = jnp.maximum(h + b1_ref[...], 0.0)
        o_ref[...] = (
            jnp.dot(h, w2_ref[...], preferred_element_type=jnp.float32)
            + b2_ref[...]
        )

    return pl.pallas_call(
        body,
        grid=(N // ROWS_BLK,),
        in_specs=[
            pl.BlockSpec((ROWS_BLK, EMBED_DIM), lambda i: (i, 0)),
            pl.BlockSpec((EMBED_DIM, HIDDEN), lambda i: (0, 0)),
            pl.BlockSpec((1, HIDDEN), lambda i: (0, 0)),
            pl.BlockSpec((HIDDEN, HIDDEN), lambda i: (0, 0)),
            pl.BlockSpec((1, HIDDEN), lambda i: (0, 0)),
        ],
        out_specs=pl.BlockSpec((ROWS_BLK, HIDDEN), lambda i: (i, 0)),
        out_shape=jax.ShapeDtypeStruct((N, HIDDEN), jnp.float32),
    )(agg, W1, b1, W2, b2)


def kernel(kg_spatial_matrix, rel_table, W1, b1, W2, b2):
    table = rel_table.at[0].set(0.0)
    idx2d = kg_spatial_matrix.reshape(N // R_CHUNK, IDX_PER_CHUNK)
    agg = _sc_gather_mean(idx2d, table)
    out = _mlp(agg, W1, b1.reshape(1, HIDDEN), W2, b2.reshape(1, HIDDEN))
    return out.reshape(B, S, HIDDEN)
